# baseline (speedup 1.0000x reference)
import jax
import jax.numpy as jnp
from jax.experimental import pallas as pl
from jax.experimental.pallas import tpu as pltpu


def _fused_kernel(x_hbm, w_ref, nb_ref, o_ref, m_ref, x_buf, sem):
    @pl.when(pl.program_id(0) == 0)
    def _():
        cp = pltpu.make_async_copy(x_hbm, x_buf, sem)
        cp.start()
        cp.wait()
        m_ref[...] = jnp.dot(
            x_buf[...], w_ref[...], preferred_element_type=jnp.float32
        )

    acc = jnp.dot(nb_ref[...], m_ref[...],
                  preferred_element_type=jnp.float32)
    o_ref[...] = jnp.maximum(acc, 0.0)


def kernel(x_source, neighborhood, weight, weight2, weight3):
    n, d_in = x_source.shape
    d_out = weight.shape[1]

    bm = 400
    out = pl.pallas_call(
        _fused_kernel,
        grid=(n // bm,),
        in_specs=[
            pl.BlockSpec(memory_space=pl.ANY),
            pl.BlockSpec((d_in, d_out), lambda i: (0, 0)),
            pl.BlockSpec((bm, n), lambda i: (i, 0)),
        ],
        out_specs=pl.BlockSpec((bm, d_out), lambda i: (i, 0)),
        out_shape=jax.ShapeDtypeStruct((n, d_out), jnp.float32),
        scratch_shapes=[
            pltpu.VMEM((n, d_out), jnp.float32),
            pltpu.VMEM((n, d_in), jnp.float32),
            pltpu.SemaphoreType.DMA,
        ],
        compiler_params=pltpu.CompilerParams(
            dimension_semantics=("arbitrary",),
        ),
    )(x_source, weight, neighborhood)
    return out


# speedup vs baseline: 1.0338x; 1.0338x over previous
import jax
import jax.numpy as jnp
from jax.experimental import pallas as pl
from jax.experimental.pallas import tpu as pltpu


def _fused_kernel(x_ref, w_ref, nb_ref, o_ref, m_ref):
    @pl.when(pl.program_id(0) == 0)
    def _():
        m_ref[...] = jnp.dot(
            x_ref[...], w_ref[...], preferred_element_type=jnp.float32
        )

    acc = jnp.dot(nb_ref[...], m_ref[...],
                  preferred_element_type=jnp.float32)
    o_ref[...] = jnp.maximum(acc, 0.0)


def kernel(x_source, neighborhood, weight, weight2, weight3):
    n, d_in = x_source.shape
    d_out = weight.shape[1]

    bm = 400
    out = pl.pallas_call(
        _fused_kernel,
        grid=(n // bm,),
        in_specs=[
            pl.BlockSpec((n, d_in), lambda i: (0, 0)),
            pl.BlockSpec((d_in, d_out), lambda i: (0, 0)),
            pl.BlockSpec((bm, n), lambda i: (i, 0)),
        ],
        out_specs=pl.BlockSpec((bm, d_out), lambda i: (i, 0)),
        out_shape=jax.ShapeDtypeStruct((n, d_out), jnp.float32),
        scratch_shapes=[pltpu.VMEM((n, d_out), jnp.float32)],
        compiler_params=pltpu.CompilerParams(
            dimension_semantics=("arbitrary",),
        ),
    )(x_source, weight, neighborhood)
    return out
